# baseline (speedup 1.0000x reference)
import jax
import jax.numpy as jnp
from jax.experimental import pallas as pl

N_ETAS = 8
TN = 512


def _compose_kernel(wshT_ref, wpr_ref, wb_ref, bb_ref, bc_ref, v_ref):
    v = wshT_ref[0] + jax.lax.dot_general(
        wpr_ref[0], wb_ref[...], (((0,), (0,)), ((), ())),
        preferred_element_type=jnp.float32)
    beff = jax.lax.dot_general(
        wpr_ref[0], bb_ref[...], (((0,), (0,)), ((), ())),
        preferred_element_type=jnp.float32) + bc_ref[0]
    v_ref[0] = jnp.concatenate([v, beff], axis=1)


def _fused_kernel(xq_ref, xs_ref, xt_ref, eta_ref, wall_ref, out_ref):
    tn = xq_ref.shape[1]
    X = jnp.concatenate(
        [xq_ref[...], xs_ref[...], xt_ref[...],
         jnp.ones((1, tn), jnp.bfloat16)], axis=0)
    Rall = jax.lax.dot_general(
        wall_ref[...], X, (((1,), (0,)), ((), ())),
        preferred_element_type=jnp.float32)
    P = Rall[:128]
    R = [Rall[128 * (e + 1):128 * (e + 2)] for e in range(N_ETAS)]
    eta = eta_ref[0]
    b0 = (eta & 1) == 1
    b1 = (eta & 2) == 2
    b2 = (eta & 4) == 4
    t0 = jnp.where(b0, R[1], R[0])
    t1 = jnp.where(b0, R[3], R[2])
    t2 = jnp.where(b0, R[5], R[4])
    t3 = jnp.where(b0, R[7], R[6])
    u0 = jnp.where(b1, t1, t0)
    u1 = jnp.where(b1, t3, t2)
    out_ref[...] = P + jnp.where(b2, u1, u0)


def kernel(x_QT, x_SLI, x_SST, eta, W_base_QT, b_base_QT, W_base_SLI,
           b_base_SLI, W_ratio_QT, b_ratio_QT, W_ratio_SLI, b_ratio_SLI):
    nz, h, w = x_QT.shape
    N = h * w
    E, FEAT, _ = W_ratio_QT.shape
    C = 2 * nz
    K = 3 * nz + 1
    xq = x_QT.reshape(nz, N).astype(jnp.bfloat16)
    xs = x_SLI.reshape(nz, N).astype(jnp.bfloat16)
    xt = x_SST.reshape(nz, N).astype(jnp.bfloat16)
    T = N // TN
    eta3 = eta.reshape(T, 1, TN).astype(jnp.int32)

    perm = jnp.concatenate([jnp.arange(2 * nz, 3 * nz),
                            jnp.arange(3 * nz, 4 * nz),
                            jnp.arange(nz, 2 * nz)])
    wshT = jnp.concatenate([W_ratio_QT[:, perm, :],
                            W_ratio_SLI[:, perm, :]],
                           axis=2).swapaxes(1, 2)
    z = jnp.zeros((E, nz, nz), jnp.float32)
    top = jnp.concatenate([W_ratio_QT[:, :nz, :], z], axis=2)
    bot = jnp.concatenate([z, W_ratio_SLI[:, :nz, :]], axis=2)
    wpr = jnp.concatenate([top, bot], axis=1)
    bc = jnp.concatenate([b_ratio_QT, b_ratio_SLI],
                         axis=1)[:, :, None]
    wb = jnp.concatenate([W_base_QT, W_base_SLI], axis=0)
    bb = jnp.concatenate([b_base_QT, b_base_SLI])[:, None]

    vall = pl.pallas_call(
        _compose_kernel,
        grid=(E,),
        in_specs=[
            pl.BlockSpec((1, C, 3 * nz), lambda e: (e, 0, 0)),
            pl.BlockSpec((1, C, C), lambda e: (e, 0, 0)),
            pl.BlockSpec((C, 3 * nz), lambda e: (0, 0)),
            pl.BlockSpec((C, 1), lambda e: (0, 0)),
            pl.BlockSpec((1, C, 1), lambda e: (e, 0, 0)),
        ],
        out_specs=pl.BlockSpec((1, C, K), lambda e: (e, 0, 0)),
        out_shape=jax.ShapeDtypeStruct((E, C, K), jnp.float32),
    )(wshT, wpr, wb, bb, bc)

    wb_aug = jnp.concatenate([wb, bb], axis=1)
    wall = jnp.concatenate([wb_aug[None], vall],
                           axis=0).reshape((E + 1) * C, K).astype(jnp.bfloat16)

    out = pl.pallas_call(
        _fused_kernel,
        grid=(T,),
        in_specs=[
            pl.BlockSpec((nz, TN), lambda t: (0, t)),
            pl.BlockSpec((nz, TN), lambda t: (0, t)),
            pl.BlockSpec((nz, TN), lambda t: (0, t)),
            pl.BlockSpec((1, 1, TN), lambda t: (t, 0, 0)),
            pl.BlockSpec(((E + 1) * C, K), lambda t: (0, 0)),
        ],
        out_specs=pl.BlockSpec((C, TN), lambda t: (0, t)),
        out_shape=jax.ShapeDtypeStruct((C, N), jnp.float32),
    )(xq, xs, xt, eta3, wall)

    return out.reshape(2, nz, h, w)


# speedup vs baseline: 1.1990x; 1.1990x over previous
import jax
import jax.numpy as jnp
from jax.experimental import pallas as pl

N_ETAS = 8
TN = 512


def _compose_kernel(wq_ref, ws_ref, brq_ref, brs_ref,
                    wbq_ref, wbs_ref, bbq_ref, bbs_ref, wall_ref):
    nz = 64
    wall_ref[:, 0:nz] = jnp.concatenate(
        [wbq_ref[...].T, bbq_ref[...].T], axis=0).astype(jnp.bfloat16)
    wall_ref[:, nz:2 * nz] = jnp.concatenate(
        [wbs_ref[...].T, bbs_ref[...].T], axis=0).astype(jnp.bfloat16)

    def expert_cols(wr, br, wb, bb, e):
        w = wr[e]
        sh = jnp.concatenate([w[2 * nz:3 * nz], w[3 * nz:], w[nz:2 * nz]],
                             axis=0)
        v = sh + jax.lax.dot_general(
            wb, w[:nz], (((0,), (0,)), ((), ())),
            preferred_element_type=jnp.float32)
        beff = jax.lax.dot_general(
            bb, w[:nz], (((0,), (0,)), ((), ())),
            preferred_element_type=jnp.float32) + br[e:e + 1]
        return jnp.concatenate([v, beff], axis=0)

    for e in range(N_ETAS):
        base = 2 * nz * (e + 1)
        wall_ref[:, base:base + nz] = expert_cols(
            wq_ref[...], brq_ref[...], wbq_ref[...], bbq_ref[...],
            e).astype(jnp.bfloat16)
        wall_ref[:, base + nz:base + 2 * nz] = expert_cols(
            ws_ref[...], brs_ref[...], wbs_ref[...], bbs_ref[...],
            e).astype(jnp.bfloat16)


def _fused_kernel(xq_ref, xs_ref, xt_ref, eta_ref, wall_ref, out_ref):
    tn = xq_ref.shape[1]
    X = jnp.concatenate(
        [xq_ref[...].astype(jnp.bfloat16),
         xs_ref[...].astype(jnp.bfloat16),
         xt_ref[...].astype(jnp.bfloat16),
         jnp.ones((1, tn), jnp.bfloat16)], axis=0)
    Rall = jax.lax.dot_general(
        wall_ref[...], X, (((0,), (0,)), ((), ())),
        preferred_element_type=jnp.float32)
    P = Rall[:128]
    R = [Rall[128 * (e + 1):128 * (e + 2)] for e in range(N_ETAS)]
    eta = eta_ref[0]
    b0 = (eta & 1) == 1
    b1 = (eta & 2) == 2
    b2 = (eta & 4) == 4
    t0 = jnp.where(b0, R[1], R[0])
    t1 = jnp.where(b0, R[3], R[2])
    t2 = jnp.where(b0, R[5], R[4])
    t3 = jnp.where(b0, R[7], R[6])
    u0 = jnp.where(b1, t1, t0)
    u1 = jnp.where(b1, t3, t2)
    out_ref[...] = P + jnp.where(b2, u1, u0)


def kernel(x_QT, x_SLI, x_SST, eta, W_base_QT, b_base_QT, W_base_SLI,
           b_base_SLI, W_ratio_QT, b_ratio_QT, W_ratio_SLI, b_ratio_SLI):
    nz, h, w = x_QT.shape
    N = h * w
    E, FEAT, _ = W_ratio_QT.shape
    C = 2 * nz
    K = 3 * nz + 1
    M = (E + 1) * C
    xq = x_QT.reshape(nz, N)
    xs = x_SLI.reshape(nz, N)
    xt = x_SST.reshape(nz, N)
    T = N // TN
    eta3 = eta.reshape(T, 1, TN).astype(jnp.int32)
    bbq = b_base_QT[:, None]
    bbs = b_base_SLI[:, None]

    full = lambda shape: pl.BlockSpec(shape, lambda *_: (0,) * len(shape))
    wall = pl.pallas_call(
        _compose_kernel,
        grid=(1,),
        in_specs=[
            full((E, FEAT, nz)), full((E, FEAT, nz)),
            full((E, nz)), full((E, nz)),
            full((nz, 3 * nz)), full((nz, 3 * nz)),
            full((nz, 1)), full((nz, 1)),
        ],
        out_specs=full((K, M)),
        out_shape=jax.ShapeDtypeStruct((K, M), jnp.bfloat16),
    )(W_ratio_QT, W_ratio_SLI, b_ratio_QT, b_ratio_SLI,
      W_base_QT, W_base_SLI, bbq, bbs)

    out = pl.pallas_call(
        _fused_kernel,
        grid=(T,),
        in_specs=[
            pl.BlockSpec((nz, TN), lambda t: (0, t)),
            pl.BlockSpec((nz, TN), lambda t: (0, t)),
            pl.BlockSpec((nz, TN), lambda t: (0, t)),
            pl.BlockSpec((1, 1, TN), lambda t: (t, 0, 0)),
            full((K, M)),
        ],
        out_specs=pl.BlockSpec((C, TN), lambda t: (0, t)),
        out_shape=jax.ShapeDtypeStruct((C, N), jnp.float32),
    )(xq, xs, xt, eta3, wall)

    return out.reshape(2, nz, h, w)


# speedup vs baseline: 1.4130x; 1.1785x over previous
import jax
import jax.numpy as jnp
from jax.experimental import pallas as pl

N_ETAS = 8
TN = 2048


def _compose_kernel(wq_ref, ws_ref, brq_ref, brs_ref,
                    wbq_ref, wbs_ref, bbq_ref, bbs_ref, wall_ref):
    nz = 64
    wall_ref[:, 0:nz] = jnp.concatenate(
        [wbq_ref[...].T, bbq_ref[...].T], axis=0).astype(jnp.bfloat16)
    wall_ref[:, nz:2 * nz] = jnp.concatenate(
        [wbs_ref[...].T, bbs_ref[...].T], axis=0).astype(jnp.bfloat16)

    def expert_cols(wr, br, wb, bb, e):
        w = wr[e]
        sh = jnp.concatenate([w[2 * nz:3 * nz], w[3 * nz:], w[nz:2 * nz]],
                             axis=0)
        v = sh + jax.lax.dot_general(
            wb, w[:nz], (((0,), (0,)), ((), ())),
            preferred_element_type=jnp.float32)
        beff = jax.lax.dot_general(
            bb, w[:nz], (((0,), (0,)), ((), ())),
            preferred_element_type=jnp.float32) + br[e:e + 1]
        return jnp.concatenate([v, beff], axis=0)

    for e in range(N_ETAS):
        base = 2 * nz * (e + 1)
        wall_ref[:, base:base + nz] = expert_cols(
            wq_ref[...], brq_ref[...], wbq_ref[...], bbq_ref[...],
            e).astype(jnp.bfloat16)
        wall_ref[:, base + nz:base + 2 * nz] = expert_cols(
            ws_ref[...], brs_ref[...], wbs_ref[...], bbs_ref[...],
            e).astype(jnp.bfloat16)


def _fused_kernel(xq_ref, xs_ref, xt_ref, eta_ref, wall_ref, out_ref):
    tn = xq_ref.shape[1]
    X = jnp.concatenate(
        [xq_ref[...].astype(jnp.bfloat16),
         xs_ref[...].astype(jnp.bfloat16),
         xt_ref[...].astype(jnp.bfloat16),
         jnp.ones((1, tn), jnp.bfloat16)], axis=0)
    Rall = jax.lax.dot_general(
        wall_ref[...], X, (((0,), (0,)), ((), ())),
        preferred_element_type=jnp.float32)
    P = Rall[:128]
    R = [Rall[128 * (e + 1):128 * (e + 2)] for e in range(N_ETAS)]
    eta = eta_ref[0]
    b0 = (eta & 1) == 1
    b1 = (eta & 2) == 2
    b2 = (eta & 4) == 4
    t0 = jnp.where(b0, R[1], R[0])
    t1 = jnp.where(b0, R[3], R[2])
    t2 = jnp.where(b0, R[5], R[4])
    t3 = jnp.where(b0, R[7], R[6])
    u0 = jnp.where(b1, t1, t0)
    u1 = jnp.where(b1, t3, t2)
    out_ref[...] = P + jnp.where(b2, u1, u0)


def kernel(x_QT, x_SLI, x_SST, eta, W_base_QT, b_base_QT, W_base_SLI,
           b_base_SLI, W_ratio_QT, b_ratio_QT, W_ratio_SLI, b_ratio_SLI):
    nz, h, w = x_QT.shape
    N = h * w
    E, FEAT, _ = W_ratio_QT.shape
    C = 2 * nz
    K = 3 * nz + 1
    M = (E + 1) * C
    xq = x_QT.reshape(nz, N)
    xs = x_SLI.reshape(nz, N)
    xt = x_SST.reshape(nz, N)
    T = N // TN
    eta3 = eta.reshape(T, 1, TN).astype(jnp.int32)
    bbq = b_base_QT[:, None]
    bbs = b_base_SLI[:, None]

    full = lambda shape: pl.BlockSpec(shape, lambda *_: (0,) * len(shape))
    wall = pl.pallas_call(
        _compose_kernel,
        grid=(1,),
        in_specs=[
            full((E, FEAT, nz)), full((E, FEAT, nz)),
            full((E, nz)), full((E, nz)),
            full((nz, 3 * nz)), full((nz, 3 * nz)),
            full((nz, 1)), full((nz, 1)),
        ],
        out_specs=full((K, M)),
        out_shape=jax.ShapeDtypeStruct((K, M), jnp.bfloat16),
    )(W_ratio_QT, W_ratio_SLI, b_ratio_QT, b_ratio_SLI,
      W_base_QT, W_base_SLI, bbq, bbs)

    out = pl.pallas_call(
        _fused_kernel,
        grid=(T,),
        in_specs=[
            pl.BlockSpec((nz, TN), lambda t: (0, t)),
            pl.BlockSpec((nz, TN), lambda t: (0, t)),
            pl.BlockSpec((nz, TN), lambda t: (0, t)),
            pl.BlockSpec((1, 1, TN), lambda t: (t, 0, 0)),
            full((K, M)),
        ],
        out_specs=pl.BlockSpec((C, TN), lambda t: (0, t)),
        out_shape=jax.ShapeDtypeStruct((C, N), jnp.float32),
    )(xq, xs, xt, eta3, wall)

    return out.reshape(2, nz, h, w)
